# baseline (speedup 1.0000x reference)
import functools

import jax
import jax.numpy as jnp
from jax import lax
from jax.experimental import pallas as pl
from jax.experimental.pallas import tpu as pltpu
from jax.experimental.pallas import tpu_sc as plsc

B, C, H, W = 16, 256, 32, 32
HW = H * W
N = B * HW
K = 1024
D = C
L = 16

NC = 2
NS = 16
NW = NC * NS
CPT = C // NW


def _tc_body(emb_ref, z_ref, idx_ref, stats_ref, counts_ref, acc_ref):
    b = pl.program_id(0)
    emb = emb_ref[...]
    zb = z_ref[0]
    s_t = jax.lax.dot_general(
        emb, zb, (((1,), (0,)), ((), ())),
        preferred_element_type=jnp.float32,
        precision=lax.Precision.DEFAULT,
    )
    enorm = jnp.sum(emb * emb, axis=1, keepdims=True)
    znorm = jnp.sum(zb * zb, axis=0, keepdims=True)
    dist_t = (znorm - 2.0 * s_t) + enorm
    m = jnp.min(dist_t, axis=0, keepdims=True)
    ks = lax.broadcasted_iota(jnp.int32, (K, HW), 0)
    eq = dist_t == m
    idx = jnp.min(jnp.where(eq, ks, K), axis=0).astype(jnp.int32)
    idx_ref[0, 0, :] = idx

    @pl.when(b == 0)
    def _init():
        counts_ref[...] = jnp.zeros_like(counts_ref)
        acc_ref[0] = 0.0

    ones = jnp.ones((HW, 1), jnp.float32)
    counts_ref[...] += jax.lax.dot_general(
        eq.astype(jnp.float32), ones, (((1,), (0,)), ((), ())),
        preferred_element_type=jnp.float32)
    esum = jnp.sum(emb, axis=0, keepdims=True)
    zsum = jnp.sum(zb, axis=1, keepdims=True)
    cross = jax.lax.dot_general(
        esum, zsum, (((1,), (0,)), ((), ())),
        preferred_element_type=jnp.float32,
        precision=lax.Precision.HIGHEST)
    acc_ref[0] += (K * jnp.sum(znorm) + HW * jnp.sum(enorm)
                   - 2.0 * cross[0, 0])

    @pl.when(b == B - 1)
    def _finalize():
        e_mean = counts_ref[...] * (1.0 / N)
        ent = jnp.sum(e_mean * jnp.log(e_mean + 1e-10))
        stats_ref[0] = jnp.exp(-ent)
        stats_ref[1] = acc_ref[0] * (1.0 / (N * K))


_tc_call = pl.pallas_call(
    _tc_body,
    grid=(B,),
    in_specs=[
        pl.BlockSpec((K, D), lambda b: (0, 0)),
        pl.BlockSpec((1, C, HW), lambda b: (b, 0, 0)),
    ],
    out_specs=[
        pl.BlockSpec((1, 1, HW), lambda b: (b, 0, 0)),
        pl.BlockSpec(memory_space=pltpu.SMEM),
    ],
    out_shape=[
        jax.ShapeDtypeStruct((B, 1, HW), jnp.int32),
        jax.ShapeDtypeStruct((2,), jnp.float32),
    ],
    scratch_shapes=[
        pltpu.VMEM((K, 1), jnp.float32),
        pltpu.SMEM((2,), jnp.float32),
    ],
)


def _sc_zq_body(embt_hbm, idx_hbm, out_hbm, embt_v, idx_v, ob0, ob1, so0, so1):
    c = lax.axis_index("c")
    s = lax.axis_index("s")
    wid = s * NC + c
    crow = wid * CPT
    pltpu.sync_copy(embt_hbm.at[pl.ds(crow * K, CPT * K)], embt_v)
    pltpu.sync_copy(idx_hbm, idx_v)

    def fill(b, ob):
        @pl.loop(0, HW // L)
        def _g(g):
            iv = idx_v[pl.ds(b * HW + g * L, L)]
            for ci in range(CPT):
                ob[pl.ds(ci * HW + g * L, L)] = plsc.load_gather(
                    embt_v, [iv + (ci * K)])

    @pl.loop(0, B // 2)
    def _k(k):
        for par, ob, so in ((0, ob0, so0), (1, ob1, so1)):
            b = 2 * k + par

            @pl.when(k > 0)
            def _wait_prev():
                pltpu.make_async_copy(
                    ob, out_hbm.at[0].at[pl.ds(crow * HW, CPT * HW)], so).wait()

            fill(b, ob)
            pltpu.async_copy(
                ob, out_hbm.at[b].at[pl.ds(crow * HW, CPT * HW)], so)

    pltpu.make_async_copy(
        ob0, out_hbm.at[0].at[pl.ds(crow * HW, CPT * HW)], so0).wait()
    pltpu.make_async_copy(
        ob1, out_hbm.at[1].at[pl.ds(crow * HW, CPT * HW)], so1).wait()


@functools.lru_cache(maxsize=1)
def _make_sc_zq():
    return pl.kernel(
        _sc_zq_body,
        out_type=jax.ShapeDtypeStruct((B, C * HW), jnp.float32),
        mesh=plsc.VectorSubcoreMesh(
            core_axis_name="c", subcore_axis_name="s",
            num_cores=NC, num_subcores=NS),
        scratch_types=[
            pltpu.VMEM((CPT * K,), jnp.float32),
            pltpu.VMEM((N,), jnp.int32),
            pltpu.VMEM((CPT * HW,), jnp.float32),
            pltpu.VMEM((CPT * HW,), jnp.float32),
            pltpu.SemaphoreType.DMA,
            pltpu.SemaphoreType.DMA,
        ],
        compiler_params=pltpu.CompilerParams(needs_layout_passes=False),
    )


def kernel(z, embedding):
    zs = z.reshape(B, C, HW)
    idx3, stats = _tc_call(embedding, zs)
    embt = embedding.T.reshape(D * K)
    zq = _make_sc_zq()(embt, idx3.reshape(N))
    z_q = zq.reshape(B, C, H, W)
    loss = jnp.zeros((), jnp.float32)
    indices = idx3.reshape(B, H, W)
    return (z_q, loss, stats[0], indices, stats[1])
